# Initial kernel scaffold; baseline (speedup 1.0000x reference)
import jax
import jax.numpy as jnp
from jax.experimental import pallas as pl


def kernel(data_x, edge_index, W1, alpha1, W2, alpha2):
    raise NotImplementedError("write your pallas kernel here")


# speedup vs baseline: 14.2518x; 14.2518x over previous
import functools

import jax
import jax.numpy as jnp
from jax import lax
from jax.experimental import pallas as pl
from jax.experimental.pallas import tpu as pltpu
from jax.experimental.pallas import tpu_sc as plsc

_N = 10000
_E = 320000
_F = 128
_NC = 2
_NS = 16
_NW = _NC * _NS
_NPAD = 10240
_TROWS = _NPAD // _NS
_EC = 128
_NCH = _E // _EC
_HW = 128

_mesh = plsc.VectorSubcoreMesh(core_axis_name="c", subcore_axis_name="s")


def _chunk_range(w):
    lo = w * _NCH // _NW
    hi = (w + 1) * _NCH // _NW
    return lo, hi


def _deg_body(col_hbm, out_hbm, cidx, ones, zeros, acc):
    c = lax.axis_index("c")
    s = lax.axis_index("s")
    w = c * _NS + s

    @pl.loop(0, _EC)
    def _(i):
        @pl.loop(0, _HW, step=16)
        def _(j):
            ones[i, pl.ds(j, 16)] = jnp.ones((16,), jnp.float32)
            zeros[i, pl.ds(j, 16)] = jnp.zeros((16,), jnp.float32)

    @pl.loop(0, _TROWS // _EC)
    def _(i):
        pltpu.sync_copy(zeros, acc.at[pl.ds(s * _TROWS + i * _EC, _EC)])

    plsc.subcore_barrier()

    lo, hi = _chunk_range(w)

    @pl.loop(lo, hi)
    def _(j):
        pltpu.sync_copy(col_hbm.at[pl.ds(j, 1)], cidx)
        pltpu.sync_copy(ones, acc.at[cidx.at[0]], add=True)

    plsc.subcore_barrier()
    pltpu.sync_copy(acc.at[pl.ds(s * _TROWS, _TROWS)],
                    out_hbm.at[pl.ds(c * _NPAD + s * _TROWS, _TROWS)])


def _gs_body(y_hbm, row_hbm, col_hbm, out_hbm, ridx, cidx, rows, acc):
    c = lax.axis_index("c")
    s = lax.axis_index("s")
    w = c * _NS + s

    @pl.loop(0, _EC)
    def _(i):
        @pl.loop(0, _F, step=16)
        def _(j):
            rows[i, pl.ds(j, 16)] = jnp.zeros((16,), jnp.float32)

    @pl.loop(0, _TROWS // _EC)
    def _(i):
        pltpu.sync_copy(rows, acc.at[pl.ds(s * _TROWS + i * _EC, _EC)])

    plsc.subcore_barrier()

    lo, hi = _chunk_range(w)

    @pl.loop(lo, hi)
    def _(j):
        pltpu.sync_copy(row_hbm.at[pl.ds(j, 1)], ridx)
        pltpu.sync_copy(col_hbm.at[pl.ds(j, 1)], cidx)
        pltpu.sync_copy(y_hbm.at[ridx.at[0]], rows)
        pltpu.sync_copy(rows, acc.at[cidx.at[0]], add=True)

    plsc.subcore_barrier()
    pltpu.sync_copy(acc.at[pl.ds(s * _TROWS, _TROWS)],
                    out_hbm.at[pl.ds(c * _NPAD + s * _TROWS, _TROWS)])


def _build_deg(**kw):
    return pl.kernel(
        _deg_body,
        out_type=jax.ShapeDtypeStruct((_NC * _NPAD, _HW), jnp.float32),
        mesh=_mesh,
        scratch_types=[
            pltpu.VMEM((1, _EC), jnp.int32),
            pltpu.VMEM((_EC, _HW), jnp.float32),
            pltpu.VMEM((_EC, _HW), jnp.float32),
            pltpu.VMEM_SHARED((_NPAD, _HW), jnp.float32),
        ],
        **kw,
    )


def _build_gs(**kw):
    return pl.kernel(
        _gs_body,
        out_type=jax.ShapeDtypeStruct((_NC * _NPAD, _F), jnp.float32),
        mesh=_mesh,
        scratch_types=[
            pltpu.VMEM((1, _EC), jnp.int32),
            pltpu.VMEM((1, _EC), jnp.int32),
            pltpu.VMEM((_EC, _F), jnp.float32),
            pltpu.VMEM_SHARED((_NPAD, _F), jnp.float32),
        ],
        **kw,
    )


_deg_kernel = _build_deg()
_gather_scatter = _build_gs()

_BR = 1000


def _dinv_of(d):
    return lax.rsqrt(d[:, 0:1] + d[:, 1:2] + 1.0)


def _dot(a, b):
    return jnp.dot(a, b, preferred_element_type=jnp.float32,
                   precision=lax.Precision.HIGHEST)


def _mm_scale(x, W, degT):
    def body(x_ref, w_ref, d_ref, o_ref):
        dinv = _dinv_of(d_ref[...])
        o_ref[...] = _dot(x_ref[...], w_ref[...]) * dinv

    return pl.pallas_call(
        body,
        grid=(_N // _BR,),
        in_specs=[
            pl.BlockSpec((_BR, _F), lambda i: (i, 0)),
            pl.BlockSpec((_F, _F), lambda i: (0, 0)),
            pl.BlockSpec((_BR, 2), lambda i: (i, 0)),
        ],
        out_specs=pl.BlockSpec((_BR, _F), lambda i: (i, 0)),
        out_shape=jax.ShapeDtypeStruct((_N, _F), jnp.float32),
    )(x, W, degT)


def _mid(acc, y, degT, W, alpha):
    def body(a_ref, y_ref, d_ref, w_ref, al_ref, o_ref):
        dinv = _dinv_of(d_ref[...])
        z = (a_ref[0] + a_ref[1] + y_ref[...]) * dinv
        z = jnp.where(z >= 0, z, al_ref[...] * z)
        o_ref[...] = _dot(z, w_ref[...]) * dinv

    return pl.pallas_call(
        body,
        grid=(_N // _BR,),
        in_specs=[
            pl.BlockSpec((2, _BR, _F), lambda i: (0, i, 0)),
            pl.BlockSpec((_BR, _F), lambda i: (i, 0)),
            pl.BlockSpec((_BR, 2), lambda i: (i, 0)),
            pl.BlockSpec((_F, _F), lambda i: (0, 0)),
            pl.BlockSpec((1, _F), lambda i: (0, 0)),
        ],
        out_specs=pl.BlockSpec((_BR, _F), lambda i: (i, 0)),
        out_shape=jax.ShapeDtypeStruct((_N, _F), jnp.float32),
    )(acc, y, degT, W, alpha)


def _fin(acc, y, degT, alpha):
    def body(a_ref, y_ref, d_ref, al_ref, o_ref):
        dinv = _dinv_of(d_ref[...])
        z = (a_ref[0] + a_ref[1] + y_ref[...]) * dinv
        o_ref[...] = jnp.where(z >= 0, z, al_ref[...] * z)

    return pl.pallas_call(
        body,
        grid=(_N // _BR,),
        in_specs=[
            pl.BlockSpec((2, _BR, _F), lambda i: (0, i, 0)),
            pl.BlockSpec((_BR, _F), lambda i: (i, 0)),
            pl.BlockSpec((_BR, 2), lambda i: (i, 0)),
            pl.BlockSpec((1, _F), lambda i: (0, 0)),
        ],
        out_specs=pl.BlockSpec((_BR, _F), lambda i: (i, 0)),
        out_shape=jax.ShapeDtypeStruct((_N, _F), jnp.float32),
    )(acc, y, degT, alpha)


def kernel(data_x, edge_index, W1, alpha1, W2, alpha2):
    row2 = edge_index[0].reshape(_NCH, _EC)
    col2 = edge_index[1].reshape(_NCH, _EC)

    degp = _deg_kernel(col2).reshape(_NC, _NPAD, _HW)
    degT = degp[:, :, 0].T

    y1 = _mm_scale(data_x, W1, degT)
    acc1 = _gather_scatter(y1, row2, col2).reshape(_NC, _NPAD, _F)
    y2 = _mid(acc1, y1, degT, W2, alpha1.reshape(1, _F))
    acc2 = _gather_scatter(y2, row2, col2).reshape(_NC, _NPAD, _F)
    return _fin(acc2, y2, degT, alpha2.reshape(1, _F))


# speedup vs baseline: 24.7559x; 1.7370x over previous
import jax
import jax.numpy as jnp
from jax import lax
from jax.experimental import pallas as pl
from jax.experimental.pallas import tpu as pltpu
from jax.experimental.pallas import tpu_sc as plsc

_N = 10000
_E = 320000
_F = 128
_NC = 2
_NS = 16
_NW = _NC * _NS
_NPAD = 10240
_TROWS = _NPAD // _NS
_EC = 125
_NCH = _E // _EC
_CPT = _NCH // _NW

_mesh = plsc.VectorSubcoreMesh(core_axis_name="c", subcore_axis_name="s")


def _fill(buf, value):
    v = jnp.full((16,), value, jnp.float32)

    @pl.loop(0, _EC)
    def _(i):
        for j in range(0, _F, 16):
            buf[i, pl.ds(j, 16)] = v


def _zero_acc_slice(src, acc, s):
    @pl.loop(0, _TROWS // _EC)
    def _(i):
        pltpu.sync_copy(src, acc.at[pl.ds(s * _TROWS + i * _EC, _EC)])

    rem = _TROWS - (_TROWS // _EC) * _EC
    if rem:
        pltpu.sync_copy(src.at[pl.ds(0, rem)],
                        acc.at[pl.ds(s * _TROWS + (_TROWS // _EC) * _EC, rem)])


def _deg_body(col_hbm, out_hbm, cidx, ones, acc):
    c = lax.axis_index("c")
    s = lax.axis_index("s")
    w = c * _NS + s
    lo = w * _CPT

    pltpu.sync_copy(col_hbm.at[pl.ds(lo, _CPT)], cidx)

    _fill(ones, 0.0)
    _zero_acc_slice(ones, acc, s)
    _fill(ones, 1.0)

    plsc.subcore_barrier()

    @pl.loop(0, _CPT)
    def _(j):
        pltpu.sync_copy(ones, acc.at[cidx.at[j, 0]], add=True)

    plsc.subcore_barrier()
    pltpu.sync_copy(acc.at[pl.ds(s * _TROWS, _TROWS)],
                    out_hbm.at[pl.ds(c * _NPAD + s * _TROWS, _TROWS)])


def _gs_body(y_hbm, row_hbm, col_hbm, out_hbm,
             ridx, cidx, rows0, rows1, acc, semg0, semg1, semc0, semc1):
    c = lax.axis_index("c")
    s = lax.axis_index("s")
    w = c * _NS + s
    lo = w * _CPT

    pltpu.sync_copy(row_hbm.at[pl.ds(lo, _CPT)], ridx)

    _fill(rows0, 0.0)
    _zero_acc_slice(rows0, acc, s)

    plsc.subcore_barrier()

    half = _CPT // 2
    pltpu.async_copy(col_hbm.at[pl.ds(lo, 1)], cidx.at[pl.ds(0, 1)], semc0)
    pltpu.async_copy(col_hbm.at[pl.ds(lo + 1, 1)], cidx.at[pl.ds(1, 1)], semc1)
    pltpu.async_copy(y_hbm.at[ridx.at[0, 0]], rows0, semg0)
    pltpu.async_copy(y_hbm.at[ridx.at[1, 0]], rows1, semg1)

    @pl.loop(0, half)
    def _(t):
        j0 = 2 * t

        pltpu.make_async_copy(y_hbm.at[ridx.at[j0, 0]], rows0, semg0).wait()
        pltpu.make_async_copy(col_hbm.at[pl.ds(lo, 1)],
                              cidx.at[pl.ds(0, 1)], semc0).wait()
        pltpu.sync_copy(rows0, acc.at[cidx.at[0, 0]], add=True)

        @pl.when(t < half - 1)
        def _():
            pltpu.async_copy(col_hbm.at[pl.ds(lo + j0 + 2, 1)],
                             cidx.at[pl.ds(0, 1)], semc0)
            pltpu.async_copy(y_hbm.at[ridx.at[j0 + 2, 0]], rows0, semg0)

        pltpu.make_async_copy(y_hbm.at[ridx.at[j0 + 1, 0]], rows1, semg1).wait()
        pltpu.make_async_copy(col_hbm.at[pl.ds(lo, 1)],
                              cidx.at[pl.ds(1, 1)], semc1).wait()
        pltpu.sync_copy(rows1, acc.at[cidx.at[1, 0]], add=True)

        @pl.when(t < half - 1)
        def _():
            pltpu.async_copy(col_hbm.at[pl.ds(lo + j0 + 3, 1)],
                             cidx.at[pl.ds(1, 1)], semc1)
            pltpu.async_copy(y_hbm.at[ridx.at[j0 + 3, 0]], rows1, semg1)

    plsc.subcore_barrier()
    pltpu.sync_copy(acc.at[pl.ds(s * _TROWS, _TROWS)],
                    out_hbm.at[pl.ds(c * _NPAD + s * _TROWS, _TROWS)])


def _build_deg(**kw):
    return pl.kernel(
        _deg_body,
        out_type=jax.ShapeDtypeStruct((_NC * _NPAD, _F), jnp.float32),
        mesh=_mesh,
        scratch_types=[
            pltpu.VMEM((_CPT, 1, _EC), jnp.int32),
            pltpu.VMEM((_EC, _F), jnp.float32),
            pltpu.VMEM_SHARED((_NPAD, _F), jnp.float32),
        ],
        **kw,
    )


def _build_gs(**kw):
    return pl.kernel(
        _gs_body,
        out_type=jax.ShapeDtypeStruct((_NC * _NPAD, _F), jnp.float32),
        mesh=_mesh,
        scratch_types=[
            pltpu.VMEM((_CPT, 1, _EC), jnp.int32),
            pltpu.VMEM((2, 1, _EC), jnp.int32),
            pltpu.VMEM((_EC, _F), jnp.float32),
            pltpu.VMEM((_EC, _F), jnp.float32),
            pltpu.VMEM_SHARED((_NPAD, _F), jnp.float32),
            pltpu.SemaphoreType.DMA,
            pltpu.SemaphoreType.DMA,
            pltpu.SemaphoreType.DMA,
            pltpu.SemaphoreType.DMA,
        ],
        **kw,
    )


_deg_kernel = _build_deg()
_gather_scatter = _build_gs()

_BR = 1000


def _dinv_of(d):
    return lax.rsqrt(d[:, 0:1] + d[:, 1:2] + 1.0)


def _dot(a, b):
    return jnp.dot(a, b, preferred_element_type=jnp.float32,
                   precision=lax.Precision.HIGHEST)


def _mm_scale(x, W, degT):
    def body(x_ref, w_ref, d_ref, o_ref):
        dinv = _dinv_of(d_ref[...])
        o_ref[...] = _dot(x_ref[...], w_ref[...]) * dinv

    return pl.pallas_call(
        body,
        grid=(_N // _BR,),
        in_specs=[
            pl.BlockSpec((_BR, _F), lambda i: (i, 0)),
            pl.BlockSpec((_F, _F), lambda i: (0, 0)),
            pl.BlockSpec((_BR, 2), lambda i: (i, 0)),
        ],
        out_specs=pl.BlockSpec((_BR, _F), lambda i: (i, 0)),
        out_shape=jax.ShapeDtypeStruct((_N, _F), jnp.float32),
    )(x, W, degT)


def _mid(acc, y, degT, W, alpha):
    def body(a_ref, y_ref, d_ref, w_ref, al_ref, o_ref):
        dinv = _dinv_of(d_ref[...])
        z = (a_ref[0] + a_ref[1] + y_ref[...]) * dinv
        z = jnp.where(z >= 0, z, al_ref[...] * z)
        o_ref[...] = _dot(z, w_ref[...]) * dinv

    return pl.pallas_call(
        body,
        grid=(_N // _BR,),
        in_specs=[
            pl.BlockSpec((2, _BR, _F), lambda i: (0, i, 0)),
            pl.BlockSpec((_BR, _F), lambda i: (i, 0)),
            pl.BlockSpec((_BR, 2), lambda i: (i, 0)),
            pl.BlockSpec((_F, _F), lambda i: (0, 0)),
            pl.BlockSpec((1, _F), lambda i: (0, 0)),
        ],
        out_specs=pl.BlockSpec((_BR, _F), lambda i: (i, 0)),
        out_shape=jax.ShapeDtypeStruct((_N, _F), jnp.float32),
    )(acc, y, degT, W, alpha)


def _fin(acc, y, degT, alpha):
    def body(a_ref, y_ref, d_ref, al_ref, o_ref):
        dinv = _dinv_of(d_ref[...])
        z = (a_ref[0] + a_ref[1] + y_ref[...]) * dinv
        o_ref[...] = jnp.where(z >= 0, z, al_ref[...] * z)

    return pl.pallas_call(
        body,
        grid=(_N // _BR,),
        in_specs=[
            pl.BlockSpec((2, _BR, _F), lambda i: (0, i, 0)),
            pl.BlockSpec((_BR, _F), lambda i: (i, 0)),
            pl.BlockSpec((_BR, 2), lambda i: (i, 0)),
            pl.BlockSpec((1, _F), lambda i: (0, 0)),
        ],
        out_specs=pl.BlockSpec((_BR, _F), lambda i: (i, 0)),
        out_shape=jax.ShapeDtypeStruct((_N, _F), jnp.float32),
    )(acc, y, degT, alpha)


def kernel(data_x, edge_index, W1, alpha1, W2, alpha2):
    row2 = edge_index[0].reshape(_NCH, 1, _EC)
    col2 = edge_index[1].reshape(_NCH, 1, _EC)

    degp = _deg_kernel(col2).reshape(_NC, _NPAD, _F)
    degT = degp[:, :, 0].T

    y1 = _mm_scale(data_x, W1, degT)
    acc1 = _gather_scatter(y1, row2, col2).reshape(_NC, _NPAD, _F)
    y2 = _mid(acc1, y1, degT, W2, alpha1.reshape(1, _F))
    acc2 = _gather_scatter(y2, row2, col2).reshape(_NC, _NPAD, _F)
    return _fin(acc2, y2, degT, alpha2.reshape(1, _F))


# speedup vs baseline: 33.6940x; 1.3610x over previous
import dataclasses

import jax
import jax.numpy as jnp
from jax import lax
from jax.experimental import pallas as pl
from jax.experimental.pallas import tpu as pltpu
from jax.experimental.pallas import tpu_sc as plsc

_N = 10000
_E = 320000
_F = 128
_NC = 2
_NS = 16
_NW = _NC * _NS
_NPAD = 10240
_TROWS = _NPAD // _NS
_EC = 125
_NCH = _E // _EC
_CPT = _NCH // _NW

_mesh = plsc.VectorSubcoreMesh(core_axis_name="c", subcore_axis_name="s")


def _fill(buf, value):
    v = jnp.full((16,), value, jnp.float32)

    @pl.loop(0, _EC)
    def _(i):
        for j in range(0, _F, 16):
            buf[i, pl.ds(j, 16)] = v


def _zero_acc_slice(src, acc, s):
    @pl.loop(0, _TROWS // _EC)
    def _(i):
        pltpu.sync_copy(src, acc.at[pl.ds(s * _TROWS + i * _EC, _EC)])

    rem = _TROWS - (_TROWS // _EC) * _EC
    if rem:
        pltpu.sync_copy(src.at[pl.ds(0, rem)],
                        acc.at[pl.ds(s * _TROWS + (_TROWS // _EC) * _EC, rem)])


_EPT = _E // _NW
_HR = _NPAD // _F


def _deg_body(col_hbm, out_hbm, cvm, hist, iot, acc):
    c = lax.axis_index("c")
    s = lax.axis_index("s")
    w = c * _NS + s

    @pl.loop(0, _HR)
    def _(i):
        for j in range(0, _F, 16):
            hist[i, pl.ds(j, 16)] = jnp.zeros((16,), jnp.float32)

    @pl.when(s == 0)
    def _():
        pltpu.sync_copy(hist, acc)

    for i in range(_HR // 16):
        iot[0, 0, pl.ds(i * 16, 16)] = (
            lax.iota(jnp.int32, 16) + jnp.int32(i * 16))

    pltpu.sync_copy(col_hbm.at[w], cvm)

    ones = jnp.ones((16,), jnp.float32)

    @pl.loop(0, _EPT // 16)
    def _(k):
        idxv = cvm[k, :]
        hi = lax.shift_right_logical(idxv, 7)
        lo2 = lax.bitwise_and(idxv, 127)
        plsc.addupdate_scatter(hist, [hi, lo2], ones)

    plsc.subcore_barrier()
    pltpu.sync_copy(hist, acc.at[iot.at[0, 0]], add=True)
    plsc.subcore_barrier()

    @pl.when(s == 0)
    def _():
        pltpu.sync_copy(acc, out_hbm.at[pl.ds(c * _HR, _HR)])


def _gs_body(y_hbm, row_hbm, col_hbm, out_hbm,
             ridx, cidx, rows0, rows1, acc, semg0, semg1, semc0, semc1):
    c = lax.axis_index("c")
    s = lax.axis_index("s")
    w = c * _NS + s
    lo = w * _CPT

    pltpu.sync_copy(row_hbm.at[pl.ds(lo, _CPT)], ridx)

    _fill(rows0, 0.0)
    _zero_acc_slice(rows0, acc, s)

    plsc.subcore_barrier()

    half = _CPT // 2
    pltpu.async_copy(col_hbm.at[pl.ds(lo, 1)], cidx.at[pl.ds(0, 1)], semc0)
    pltpu.async_copy(col_hbm.at[pl.ds(lo + 1, 1)], cidx.at[pl.ds(1, 1)], semc1)
    pltpu.async_copy(y_hbm.at[ridx.at[0, 0]], rows0, semg0)
    pltpu.async_copy(y_hbm.at[ridx.at[1, 0]], rows1, semg1)

    @pl.loop(0, half)
    def _(t):
        j0 = 2 * t

        pltpu.make_async_copy(y_hbm.at[ridx.at[j0, 0]], rows0, semg0).wait()
        pltpu.make_async_copy(col_hbm.at[pl.ds(lo, 1)],
                              cidx.at[pl.ds(0, 1)], semc0).wait()
        pltpu.sync_copy(rows0, acc.at[cidx.at[0, 0]], add=True)

        @pl.when(t < half - 1)
        def _():
            pltpu.async_copy(col_hbm.at[pl.ds(lo + j0 + 2, 1)],
                             cidx.at[pl.ds(0, 1)], semc0)
            pltpu.async_copy(y_hbm.at[ridx.at[j0 + 2, 0]], rows0, semg0)

        pltpu.make_async_copy(y_hbm.at[ridx.at[j0 + 1, 0]], rows1, semg1).wait()
        pltpu.make_async_copy(col_hbm.at[pl.ds(lo, 1)],
                              cidx.at[pl.ds(1, 1)], semc1).wait()
        pltpu.sync_copy(rows1, acc.at[cidx.at[1, 0]], add=True)

        @pl.when(t < half - 1)
        def _():
            pltpu.async_copy(col_hbm.at[pl.ds(lo + j0 + 3, 1)],
                             cidx.at[pl.ds(1, 1)], semc1)
            pltpu.async_copy(y_hbm.at[ridx.at[j0 + 3, 0]], rows1, semg1)

    plsc.subcore_barrier()
    pltpu.sync_copy(acc.at[pl.ds(s * _TROWS, _TROWS)],
                    out_hbm.at[pl.ds(c * _NPAD + s * _TROWS, _TROWS)])


def _build_deg(**kw):
    cp = pltpu.CompilerParams()
    if "needs_layout_passes" in pltpu.CompilerParams.__dataclass_fields__:
        cp = dataclasses.replace(cp, needs_layout_passes=False)
    return pl.kernel(
        _deg_body,
        out_type=jax.ShapeDtypeStruct((_NC * _HR, _F), jnp.float32),
        mesh=_mesh,
        compiler_params=cp,
        scratch_types=[
            pltpu.VMEM((_EPT // 16, 16), jnp.int32),
            pltpu.VMEM((_HR, _F), jnp.float32),
            pltpu.VMEM((1, 1, _HR), jnp.int32),
            pltpu.VMEM_SHARED((_HR, _F), jnp.float32),
        ],
        **kw,
    )


def _build_gs(**kw):
    return pl.kernel(
        _gs_body,
        out_type=jax.ShapeDtypeStruct((_NC * _NPAD, _F), jnp.float32),
        mesh=_mesh,
        scratch_types=[
            pltpu.VMEM((_CPT, 1, _EC), jnp.int32),
            pltpu.VMEM((2, 1, _EC), jnp.int32),
            pltpu.VMEM((_EC, _F), jnp.float32),
            pltpu.VMEM((_EC, _F), jnp.float32),
            pltpu.VMEM_SHARED((_NPAD, _F), jnp.float32),
            pltpu.SemaphoreType.DMA,
            pltpu.SemaphoreType.DMA,
            pltpu.SemaphoreType.DMA,
            pltpu.SemaphoreType.DMA,
        ],
        **kw,
    )


_deg_kernel = _build_deg()
_gather_scatter = _build_gs()

_BR = 1000


def _dinv_of(d):
    return lax.rsqrt(d[:, 0:1] + d[:, 1:2] + 1.0)


def _dot(a, b):
    return jnp.dot(a, b, preferred_element_type=jnp.float32,
                   precision=lax.Precision.HIGHEST)


def _mm_scale(x, W, degT):
    def body(x_ref, w_ref, d_ref, o_ref):
        dinv = _dinv_of(d_ref[...])
        o_ref[...] = _dot(x_ref[...], w_ref[...]) * dinv

    return pl.pallas_call(
        body,
        grid=(_N // _BR,),
        in_specs=[
            pl.BlockSpec((_BR, _F), lambda i: (i, 0)),
            pl.BlockSpec((_F, _F), lambda i: (0, 0)),
            pl.BlockSpec((_BR, 2), lambda i: (i, 0)),
        ],
        out_specs=pl.BlockSpec((_BR, _F), lambda i: (i, 0)),
        out_shape=jax.ShapeDtypeStruct((_N, _F), jnp.float32),
    )(x, W, degT)


def _mid(acc, y, degT, W, alpha):
    def body(a_ref, y_ref, d_ref, w_ref, al_ref, o_ref):
        dinv = _dinv_of(d_ref[...])
        z = (a_ref[0] + a_ref[1] + y_ref[...]) * dinv
        z = jnp.where(z >= 0, z, al_ref[...] * z)
        o_ref[...] = _dot(z, w_ref[...]) * dinv

    return pl.pallas_call(
        body,
        grid=(_N // _BR,),
        in_specs=[
            pl.BlockSpec((2, _BR, _F), lambda i: (0, i, 0)),
            pl.BlockSpec((_BR, _F), lambda i: (i, 0)),
            pl.BlockSpec((_BR, 2), lambda i: (i, 0)),
            pl.BlockSpec((_F, _F), lambda i: (0, 0)),
            pl.BlockSpec((1, _F), lambda i: (0, 0)),
        ],
        out_specs=pl.BlockSpec((_BR, _F), lambda i: (i, 0)),
        out_shape=jax.ShapeDtypeStruct((_N, _F), jnp.float32),
    )(acc, y, degT, W, alpha)


def _fin(acc, y, degT, alpha):
    def body(a_ref, y_ref, d_ref, al_ref, o_ref):
        dinv = _dinv_of(d_ref[...])
        z = (a_ref[0] + a_ref[1] + y_ref[...]) * dinv
        o_ref[...] = jnp.where(z >= 0, z, al_ref[...] * z)

    return pl.pallas_call(
        body,
        grid=(_N // _BR,),
        in_specs=[
            pl.BlockSpec((2, _BR, _F), lambda i: (0, i, 0)),
            pl.BlockSpec((_BR, _F), lambda i: (i, 0)),
            pl.BlockSpec((_BR, 2), lambda i: (i, 0)),
            pl.BlockSpec((1, _F), lambda i: (0, 0)),
        ],
        out_specs=pl.BlockSpec((_BR, _F), lambda i: (i, 0)),
        out_shape=jax.ShapeDtypeStruct((_N, _F), jnp.float32),
    )(acc, y, degT, alpha)


def kernel(data_x, edge_index, W1, alpha1, W2, alpha2):
    row2 = edge_index[0].reshape(_NCH, 1, _EC)
    col2 = edge_index[1].reshape(_NCH, 1, _EC)
    col3 = edge_index[1].reshape(_NW, _EPT // 16, 16)

    degT = _deg_kernel(col3).reshape(_NC, _NPAD).T

    y1 = _mm_scale(data_x, W1, degT)
    acc1 = _gather_scatter(y1, row2, col2).reshape(_NC, _NPAD, _F)
    y2 = _mid(acc1, y1, degT, W2, alpha1.reshape(1, _F))
    acc2 = _gather_scatter(y2, row2, col2).reshape(_NC, _NPAD, _F)
    return _fin(acc2, y2, degT, alpha2.reshape(1, _F))


# speedup vs baseline: 33.7296x; 1.0011x over previous
import dataclasses

import jax
import jax.numpy as jnp
from jax import lax
from jax.experimental import pallas as pl
from jax.experimental.pallas import tpu as pltpu
from jax.experimental.pallas import tpu_sc as plsc

_N = 10000
_E = 320000
_F = 128
_NC = 2
_NS = 16
_NW = _NC * _NS
_NPAD = 10240
_TROWS = _NPAD // _NS
_EC = 125
_NCH = _E // _EC
_CPT = _NCH // _NW

_mesh = plsc.VectorSubcoreMesh(core_axis_name="c", subcore_axis_name="s")


def _fill(buf, value):
    v = jnp.full((16,), value, jnp.float32)

    @pl.loop(0, _EC)
    def _(i):
        for j in range(0, _F, 16):
            buf[i, pl.ds(j, 16)] = v


def _zero_acc_slice(src, acc, s):
    @pl.loop(0, _TROWS // _EC)
    def _(i):
        pltpu.sync_copy(src, acc.at[pl.ds(s * _TROWS + i * _EC, _EC)])

    rem = _TROWS - (_TROWS // _EC) * _EC
    if rem:
        pltpu.sync_copy(src.at[pl.ds(0, rem)],
                        acc.at[pl.ds(s * _TROWS + (_TROWS // _EC) * _EC, rem)])


_EPT = _E // _NW
_HR = _NPAD // _F


def _deg_body(col_hbm, out_hbm, cvm, hist, iot, acc):
    c = lax.axis_index("c")
    s = lax.axis_index("s")
    w = c * _NS + s

    @pl.loop(0, _HR)
    def _(i):
        for j in range(0, _F, 16):
            hist[i, pl.ds(j, 16)] = jnp.zeros((16,), jnp.float32)

    @pl.when(s == 0)
    def _():
        pltpu.sync_copy(hist, acc)

    for i in range(_HR // 16):
        iot[0, 0, pl.ds(i * 16, 16)] = (
            lax.iota(jnp.int32, 16) + jnp.int32(i * 16))

    pltpu.sync_copy(col_hbm.at[w], cvm)

    ones = jnp.ones((16,), jnp.float32)

    @pl.loop(0, _EPT // 16 // 5)
    def _(k):
        for u in range(5):
            idxv = cvm[k * 5 + u, :]
            hi = lax.shift_right_logical(idxv, 7)
            lo2 = lax.bitwise_and(idxv, 127)
            plsc.addupdate_scatter(hist, [hi, lo2], ones)

    plsc.subcore_barrier()
    pltpu.sync_copy(hist, acc.at[iot.at[0, 0]], add=True)
    plsc.subcore_barrier()

    @pl.when(s == 0)
    def _():
        pltpu.sync_copy(acc, out_hbm.at[pl.ds(c * _HR, _HR)])


def _gs_body(y_hbm, row_hbm, col_hbm, out_hbm,
             ridx, cstg, rows0, rows1, acc, sem0, sem1, semc0, semc1):
    c = lax.axis_index("c")
    s = lax.axis_index("s")
    w = c * _NS + s
    lo = w * _CPT

    pltpu.sync_copy(row_hbm.at[pl.ds(lo, _CPT)], ridx)

    _fill(rows0, 0.0)
    _zero_acc_slice(rows0, acc, s)

    plsc.subcore_barrier()

    half = _CPT // 2
    pltpu.async_copy(col_hbm.at[pl.ds(lo, 1)], cstg.at[pl.ds(0, 1)], semc0)
    pltpu.async_copy(col_hbm.at[pl.ds(lo + 1, 1)], cstg.at[pl.ds(1, 1)], semc1)
    pltpu.async_copy(y_hbm.at[ridx.at[0, 0]], rows0, sem0)
    pltpu.async_copy(y_hbm.at[ridx.at[1, 0]], rows1, sem1)

    @pl.loop(0, half)
    def _(t):
        j0 = 2 * t

        pltpu.make_async_copy(y_hbm.at[ridx.at[j0, 0]], rows0, sem0).wait()
        pltpu.make_async_copy(col_hbm.at[pl.ds(lo, 1)],
                              cstg.at[pl.ds(0, 1)], semc0).wait()
        pltpu.sync_copy(rows0, acc.at[cstg.at[0, 0]], add=True)

        @pl.when(t < half - 1)
        def _():
            pltpu.async_copy(col_hbm.at[pl.ds(lo + j0 + 2, 1)],
                             cstg.at[pl.ds(0, 1)], semc0)
            pltpu.async_copy(y_hbm.at[ridx.at[j0 + 2, 0]], rows0, sem0)

        pltpu.make_async_copy(y_hbm.at[ridx.at[j0 + 1, 0]], rows1, sem1).wait()
        pltpu.make_async_copy(col_hbm.at[pl.ds(lo, 1)],
                              cstg.at[pl.ds(1, 1)], semc1).wait()
        pltpu.sync_copy(rows1, acc.at[cstg.at[1, 0]], add=True)

        @pl.when(t < half - 1)
        def _():
            pltpu.async_copy(col_hbm.at[pl.ds(lo + j0 + 3, 1)],
                             cstg.at[pl.ds(1, 1)], semc1)
            pltpu.async_copy(y_hbm.at[ridx.at[j0 + 3, 0]], rows1, sem1)

    plsc.subcore_barrier()
    pltpu.sync_copy(acc.at[pl.ds(s * _TROWS, _TROWS)],
                    out_hbm.at[pl.ds(c * _NPAD + s * _TROWS, _TROWS)])


def _build_deg(**kw):
    cp = pltpu.CompilerParams()
    if "needs_layout_passes" in pltpu.CompilerParams.__dataclass_fields__:
        cp = dataclasses.replace(cp, needs_layout_passes=False)
    return pl.kernel(
        _deg_body,
        out_type=jax.ShapeDtypeStruct((_NC * _HR, _F), jnp.float32),
        mesh=_mesh,
        compiler_params=cp,
        scratch_types=[
            pltpu.VMEM((_EPT // 16, 16), jnp.int32),
            pltpu.VMEM((_HR, _F), jnp.float32),
            pltpu.VMEM((1, 1, _HR), jnp.int32),
            pltpu.VMEM_SHARED((_HR, _F), jnp.float32),
        ],
        **kw,
    )


def _build_gs(**kw):
    return pl.kernel(
        _gs_body,
        out_type=jax.ShapeDtypeStruct((_NC * _NPAD, _F), jnp.float32),
        mesh=_mesh,
        scratch_types=[
            pltpu.VMEM((_CPT, 1, _EC), jnp.int32),
            pltpu.VMEM((2, 1, _EC), jnp.int32),
            pltpu.VMEM((_EC, _F), jnp.float32),
            pltpu.VMEM((_EC, _F), jnp.float32),
            pltpu.VMEM_SHARED((_NPAD, _F), jnp.float32),
            pltpu.SemaphoreType.DMA,
            pltpu.SemaphoreType.DMA,
            pltpu.SemaphoreType.DMA,
            pltpu.SemaphoreType.DMA,
        ],
        **kw,
    )


_deg_kernel = _build_deg()
_gather_scatter = _build_gs()

_BR = 1000


def _dinv_of(d):
    return lax.rsqrt(d[:, 0:1] + d[:, 1:2] + 1.0)


def _dot(a, b):
    return jnp.dot(a, b, preferred_element_type=jnp.float32,
                   precision=lax.Precision.HIGHEST)


def _mm_scale(x, W, degT):
    def body(x_ref, w_ref, d_ref, o_ref):
        dinv = _dinv_of(d_ref[...])
        o_ref[...] = _dot(x_ref[...], w_ref[...]) * dinv

    return pl.pallas_call(
        body,
        grid=(_N // _BR,),
        in_specs=[
            pl.BlockSpec((_BR, _F), lambda i: (i, 0)),
            pl.BlockSpec((_F, _F), lambda i: (0, 0)),
            pl.BlockSpec((_BR, 2), lambda i: (i, 0)),
        ],
        out_specs=pl.BlockSpec((_BR, _F), lambda i: (i, 0)),
        out_shape=jax.ShapeDtypeStruct((_N, _F), jnp.float32),
    )(x, W, degT)


def _mid(acc, y, degT, W, alpha):
    def body(a_ref, y_ref, d_ref, w_ref, al_ref, o_ref):
        dinv = _dinv_of(d_ref[...])
        z = (a_ref[0] + a_ref[1] + y_ref[...]) * dinv
        z = jnp.where(z >= 0, z, al_ref[...] * z)
        o_ref[...] = _dot(z, w_ref[...]) * dinv

    return pl.pallas_call(
        body,
        grid=(_N // _BR,),
        in_specs=[
            pl.BlockSpec((2, _BR, _F), lambda i: (0, i, 0)),
            pl.BlockSpec((_BR, _F), lambda i: (i, 0)),
            pl.BlockSpec((_BR, 2), lambda i: (i, 0)),
            pl.BlockSpec((_F, _F), lambda i: (0, 0)),
            pl.BlockSpec((1, _F), lambda i: (0, 0)),
        ],
        out_specs=pl.BlockSpec((_BR, _F), lambda i: (i, 0)),
        out_shape=jax.ShapeDtypeStruct((_N, _F), jnp.float32),
    )(acc, y, degT, W, alpha)


def _fin(acc, y, degT, alpha):
    def body(a_ref, y_ref, d_ref, al_ref, o_ref):
        dinv = _dinv_of(d_ref[...])
        z = (a_ref[0] + a_ref[1] + y_ref[...]) * dinv
        o_ref[...] = jnp.where(z >= 0, z, al_ref[...] * z)

    return pl.pallas_call(
        body,
        grid=(_N // _BR,),
        in_specs=[
            pl.BlockSpec((2, _BR, _F), lambda i: (0, i, 0)),
            pl.BlockSpec((_BR, _F), lambda i: (i, 0)),
            pl.BlockSpec((_BR, 2), lambda i: (i, 0)),
            pl.BlockSpec((1, _F), lambda i: (0, 0)),
        ],
        out_specs=pl.BlockSpec((_BR, _F), lambda i: (i, 0)),
        out_shape=jax.ShapeDtypeStruct((_N, _F), jnp.float32),
    )(acc, y, degT, alpha)


def kernel(data_x, edge_index, W1, alpha1, W2, alpha2):
    row2 = edge_index[0].reshape(_NCH, 1, _EC)
    col2 = edge_index[1].reshape(_NCH, 1, _EC)
    col3 = edge_index[1].reshape(_NW, _EPT // 16, 16)

    degT = _deg_kernel(col3).reshape(_NC, _NPAD).T

    y1 = _mm_scale(data_x, W1, degT)
    acc1 = _gather_scatter(y1, row2, col2).reshape(_NC, _NPAD, _F)
    y2 = _mid(acc1, y1, degT, W2, alpha1.reshape(1, _F))
    acc2 = _gather_scatter(y2, row2, col2).reshape(_NC, _NPAD, _F)
    return _fin(acc2, y2, degT, alpha2.reshape(1, _F))


# speedup vs baseline: 34.1503x; 1.0125x over previous
import dataclasses

import jax
import jax.numpy as jnp
from jax import lax
from jax.experimental import pallas as pl
from jax.experimental.pallas import tpu as pltpu
from jax.experimental.pallas import tpu_sc as plsc

_N = 10000
_E = 320000
_F = 128
_NC = 2
_NS = 16
_NW = _NC * _NS
_NPAD = 10240
_TROWS = _NPAD // _NS
_EC = 125
_NCH = _E // _EC
_CPT = _NCH // _NW

_mesh = plsc.VectorSubcoreMesh(core_axis_name="c", subcore_axis_name="s")


def _fill(buf, value):
    v = jnp.full((16,), value, jnp.float32)

    @pl.loop(0, _EC)
    def _(i):
        for j in range(0, _F, 16):
            buf[i, pl.ds(j, 16)] = v


def _zero_acc_slice(src, acc, s, sem):
    for i in range(_TROWS // _EC):
        pltpu.async_copy(src, acc.at[pl.ds(s * _TROWS + i * _EC, _EC)], sem)

    rem = _TROWS - (_TROWS // _EC) * _EC
    if rem:
        pltpu.async_copy(src.at[pl.ds(0, rem)],
                         acc.at[pl.ds(s * _TROWS + (_TROWS // _EC) * _EC, rem)],
                         sem)


def _zero_acc_wait(src, acc, s, sem):
    for i in range(_TROWS // _EC):
        pltpu.make_async_copy(
            src, acc.at[pl.ds(s * _TROWS + i * _EC, _EC)], sem).wait()

    rem = _TROWS - (_TROWS // _EC) * _EC
    if rem:
        pltpu.make_async_copy(
            src.at[pl.ds(0, rem)],
            acc.at[pl.ds(s * _TROWS + (_TROWS // _EC) * _EC, rem)],
            sem).wait()


_EPT = _E // _NW
_HR = _NPAD // _F


def _deg_body(col_hbm, out_hbm, cvm, hist, iot, acc):
    c = lax.axis_index("c")
    s = lax.axis_index("s")
    w = c * _NS + s

    @pl.loop(0, _HR)
    def _(i):
        for j in range(0, _F, 16):
            hist[i, pl.ds(j, 16)] = jnp.zeros((16,), jnp.float32)

    @pl.when(s == 0)
    def _():
        pltpu.sync_copy(hist, acc)

    for i in range(_HR // 16):
        iot[0, 0, pl.ds(i * 16, 16)] = (
            lax.iota(jnp.int32, 16) + jnp.int32(i * 16))

    pltpu.sync_copy(col_hbm.at[w], cvm)

    ones = jnp.ones((16,), jnp.float32)

    @pl.loop(0, _EPT // 16 // 5)
    def _(k):
        for u in range(5):
            idxv = cvm[k * 5 + u, :]
            hi = lax.shift_right_logical(idxv, 7)
            lo2 = lax.bitwise_and(idxv, 127)
            plsc.addupdate_scatter(hist, [hi, lo2], ones)

    plsc.subcore_barrier()
    pltpu.sync_copy(hist, acc.at[iot.at[0, 0]], add=True)
    plsc.subcore_barrier()

    @pl.when(s == 0)
    def _():
        pltpu.sync_copy(acc, out_hbm.at[pl.ds(c * _HR, _HR)])


def _gs_body(y_hbm, row_hbm, col_hbm, out_hbm,
             ridx, cstg, rows0, rows1, acc, sem0, sem1, semc0, semc1, semz):
    c = lax.axis_index("c")
    s = lax.axis_index("s")
    w = c * _NS + s
    lo = w * _CPT

    pltpu.sync_copy(row_hbm.at[pl.ds(lo, _CPT)], ridx)

    _fill(rows1, 0.0)
    _zero_acc_slice(rows1, acc, s, semz)

    half = _CPT // 2
    pltpu.async_copy(col_hbm.at[pl.ds(lo, 1)], cstg.at[pl.ds(0, 1)], semc0)
    pltpu.async_copy(col_hbm.at[pl.ds(lo + 1, 1)], cstg.at[pl.ds(1, 1)], semc1)
    pltpu.async_copy(y_hbm.at[ridx.at[0, 0]], rows0, sem0)

    _zero_acc_wait(rows1, acc, s, semz)
    pltpu.async_copy(y_hbm.at[ridx.at[1, 0]], rows1, sem1)

    plsc.subcore_barrier()

    @pl.loop(0, half)
    def _(t):
        j0 = 2 * t

        pltpu.make_async_copy(y_hbm.at[ridx.at[j0, 0]], rows0, sem0).wait()
        pltpu.make_async_copy(col_hbm.at[pl.ds(lo, 1)],
                              cstg.at[pl.ds(0, 1)], semc0).wait()
        pltpu.sync_copy(rows0, acc.at[cstg.at[0, 0]], add=True)

        @pl.when(t < half - 1)
        def _():
            pltpu.async_copy(col_hbm.at[pl.ds(lo + j0 + 2, 1)],
                             cstg.at[pl.ds(0, 1)], semc0)
            pltpu.async_copy(y_hbm.at[ridx.at[j0 + 2, 0]], rows0, sem0)

        pltpu.make_async_copy(y_hbm.at[ridx.at[j0 + 1, 0]], rows1, sem1).wait()
        pltpu.make_async_copy(col_hbm.at[pl.ds(lo, 1)],
                              cstg.at[pl.ds(1, 1)], semc1).wait()
        pltpu.sync_copy(rows1, acc.at[cstg.at[1, 0]], add=True)

        @pl.when(t < half - 1)
        def _():
            pltpu.async_copy(col_hbm.at[pl.ds(lo + j0 + 3, 1)],
                             cstg.at[pl.ds(1, 1)], semc1)
            pltpu.async_copy(y_hbm.at[ridx.at[j0 + 3, 0]], rows1, sem1)

    plsc.subcore_barrier()
    pltpu.sync_copy(acc.at[pl.ds(s * _TROWS, _TROWS)],
                    out_hbm.at[pl.ds(c * _NPAD + s * _TROWS, _TROWS)])


def _build_deg(**kw):
    cp = pltpu.CompilerParams()
    if "needs_layout_passes" in pltpu.CompilerParams.__dataclass_fields__:
        cp = dataclasses.replace(cp, needs_layout_passes=False)
    return pl.kernel(
        _deg_body,
        out_type=jax.ShapeDtypeStruct((_NC * _HR, _F), jnp.float32),
        mesh=_mesh,
        compiler_params=cp,
        scratch_types=[
            pltpu.VMEM((_EPT // 16, 16), jnp.int32),
            pltpu.VMEM((_HR, _F), jnp.float32),
            pltpu.VMEM((1, 1, _HR), jnp.int32),
            pltpu.VMEM_SHARED((_HR, _F), jnp.float32),
        ],
        **kw,
    )


def _build_gs(**kw):
    return pl.kernel(
        _gs_body,
        out_type=jax.ShapeDtypeStruct((_NC * _NPAD, _F), jnp.float32),
        mesh=_mesh,
        scratch_types=[
            pltpu.VMEM((_CPT, 1, _EC), jnp.int32),
            pltpu.VMEM((2, 1, _EC), jnp.int32),
            pltpu.VMEM((_EC, _F), jnp.float32),
            pltpu.VMEM((_EC, _F), jnp.float32),
            pltpu.VMEM_SHARED((_NPAD, _F), jnp.float32),
            pltpu.SemaphoreType.DMA,
            pltpu.SemaphoreType.DMA,
            pltpu.SemaphoreType.DMA,
            pltpu.SemaphoreType.DMA,
            pltpu.SemaphoreType.DMA,
        ],
        **kw,
    )


_deg_kernel = _build_deg()
_gather_scatter = _build_gs()

_BR = 1000


def _dinv_of(d):
    return lax.rsqrt(d[:, 0:1] + d[:, 1:2] + 1.0)


def _dot(a, b):
    return jnp.dot(a, b, preferred_element_type=jnp.float32,
                   precision=lax.Precision.HIGHEST)


def _mm_scale(x, W, degT):
    def body(x_ref, w_ref, d_ref, o_ref):
        dinv = _dinv_of(d_ref[...])
        o_ref[...] = _dot(x_ref[...], w_ref[...]) * dinv

    return pl.pallas_call(
        body,
        grid=(_N // _BR,),
        in_specs=[
            pl.BlockSpec((_BR, _F), lambda i: (i, 0)),
            pl.BlockSpec((_F, _F), lambda i: (0, 0)),
            pl.BlockSpec((_BR, 2), lambda i: (i, 0)),
        ],
        out_specs=pl.BlockSpec((_BR, _F), lambda i: (i, 0)),
        out_shape=jax.ShapeDtypeStruct((_N, _F), jnp.float32),
    )(x, W, degT)


def _mid(acc, y, degT, W, alpha):
    def body(a_ref, y_ref, d_ref, w_ref, al_ref, o_ref):
        dinv = _dinv_of(d_ref[...])
        z = (a_ref[0] + a_ref[1] + y_ref[...]) * dinv
        z = jnp.where(z >= 0, z, al_ref[...] * z)
        o_ref[...] = _dot(z, w_ref[...]) * dinv

    return pl.pallas_call(
        body,
        grid=(_N // _BR,),
        in_specs=[
            pl.BlockSpec((2, _BR, _F), lambda i: (0, i, 0)),
            pl.BlockSpec((_BR, _F), lambda i: (i, 0)),
            pl.BlockSpec((_BR, 2), lambda i: (i, 0)),
            pl.BlockSpec((_F, _F), lambda i: (0, 0)),
            pl.BlockSpec((1, _F), lambda i: (0, 0)),
        ],
        out_specs=pl.BlockSpec((_BR, _F), lambda i: (i, 0)),
        out_shape=jax.ShapeDtypeStruct((_N, _F), jnp.float32),
    )(acc, y, degT, W, alpha)


def _fin(acc, y, degT, alpha):
    def body(a_ref, y_ref, d_ref, al_ref, o_ref):
        dinv = _dinv_of(d_ref[...])
        z = (a_ref[0] + a_ref[1] + y_ref[...]) * dinv
        o_ref[...] = jnp.where(z >= 0, z, al_ref[...] * z)

    return pl.pallas_call(
        body,
        grid=(_N // _BR,),
        in_specs=[
            pl.BlockSpec((2, _BR, _F), lambda i: (0, i, 0)),
            pl.BlockSpec((_BR, _F), lambda i: (i, 0)),
            pl.BlockSpec((_BR, 2), lambda i: (i, 0)),
            pl.BlockSpec((1, _F), lambda i: (0, 0)),
        ],
        out_specs=pl.BlockSpec((_BR, _F), lambda i: (i, 0)),
        out_shape=jax.ShapeDtypeStruct((_N, _F), jnp.float32),
    )(acc, y, degT, alpha)


def kernel(data_x, edge_index, W1, alpha1, W2, alpha2):
    row2 = edge_index[0].reshape(_NCH, 1, _EC)
    col2 = edge_index[1].reshape(_NCH, 1, _EC)
    col3 = edge_index[1].reshape(_NW, _EPT // 16, 16)

    degT = _deg_kernel(col3).reshape(_NC, _NPAD).T

    y1 = _mm_scale(data_x, W1, degT)
    acc1 = _gather_scatter(y1, row2, col2).reshape(_NC, _NPAD, _F)
    y2 = _mid(acc1, y1, degT, W2, alpha1.reshape(1, _F))
    acc2 = _gather_scatter(y2, row2, col2).reshape(_NC, _NPAD, _F)
    return _fin(acc2, y2, degT, alpha2.reshape(1, _F))
